# baseline (speedup 1.0000x reference)
import jax
import jax.numpy as jnp
from jax import lax
from jax.experimental import pallas as pl
from jax.experimental.pallas import tpu as pltpu
from jax.experimental.pallas import tpu_sc as plsc

B = 4
T = 8192
D = 1024
NC = 2
NS = 16
NW = NC * NS
TPW = T // NW
CH = 8
CHD = CH * D
NCH = TPW // CH
NBUF = 3
UNROLL = 8


def _sc_body(x_hbm, p_hbm, out_hbm,
             xb0, xb1, xb2, pb0, pb1, pb2,
             sx0, sx1, sx2, sp0, sp1, sp2, so0, so1, so2):
    wid = lax.axis_index("s") * NC + lax.axis_index("c")
    base = wid * TPW * D
    rings = ((xb0, pb0, sx0, sp0, so0),
             (xb1, pb1, sx1, sp1, so1),
             (xb2, pb2, sx2, sp2, so2))

    def start_in(j, r):
        xb, pb, sx, sp, _ = rings[r]
        off = base + j * CHD
        pltpu.async_copy(p_hbm.at[pl.ds(off, CHD)], pb, sp)
        pltpu.async_copy(x_hbm.at[:, pl.ds(off, CHD)], xb, sx)

    def wait_in(r):
        xb, pb, sx, sp, _ = rings[r]
        pltpu.make_async_copy(p_hbm.at[pl.ds(0, CHD)], pb, sp).wait()
        pltpu.make_async_copy(x_hbm.at[:, pl.ds(0, CHD)], xb, sx).wait()

    def start_out(j, r):
        xb, _, _, _, so = rings[r]
        off = base + j * CHD
        pltpu.async_copy(xb, out_hbm.at[:, pl.ds(off, CHD)], so)

    def wait_out(r):
        xb, _, _, _, so = rings[r]
        pltpu.make_async_copy(xb, out_hbm.at[:, pl.ds(0, CHD)], so).wait()

    def compute(r):
        xb, pb, _, _, _ = rings[r]

        @plsc.parallel_loop(0, CHD, 16, unroll=UNROLL)
        def _(o):
            pv = pb[pl.ds(o, 16)]
            for b in range(B):
                plsc.addupdate(xb.at[b, pl.ds(o, 16)], pv)

    def position(j, r, first, last):
        wait_in(r)
        compute(r)
        start_out(j, r)
        rn = (r + 2) % NBUF
        if not first:
            wait_out(rn)
        if not last:
            start_in(j + 2, rn)

    start_in(0, 0)
    start_in(1, 1)
    position(0, 0, first=True, last=False)

    def triple(t, carry):
        for k in range(3):
            j = 3 * t + 1 + k
            r = (1 + k) % NBUF
            xb, pb, sx, sp, so = rings[r]
            wait_in(r)
            compute(r)
            start_out(j, r)
            rn = (r + 2) % NBUF
            wait_out(rn)

            @pl.when(j + 2 < NCH)
            def _():
                start_in(j + 2, rn)

        return carry

    lax.fori_loop(0, (NCH - 2) // 3, triple, 0)

    jl = NCH - 1
    rl = jl % NBUF
    wait_in(rl)
    compute(rl)
    start_out(jl, rl)
    wait_out((rl + 2) % NBUF)
    wait_out(rl)


def _sc_add(x2, p1):
    mesh = plsc.VectorSubcoreMesh(core_axis_name="c", subcore_axis_name="s")
    k = pl.kernel(
        _sc_body,
        out_type=jax.ShapeDtypeStruct((B, T * D), jnp.float32),
        mesh=mesh,
        scratch_types=[
            pltpu.VMEM((B, CHD), jnp.float32),
            pltpu.VMEM((B, CHD), jnp.float32),
            pltpu.VMEM((B, CHD), jnp.float32),
            pltpu.VMEM((CHD,), jnp.float32),
            pltpu.VMEM((CHD,), jnp.float32),
            pltpu.VMEM((CHD,), jnp.float32),
            pltpu.SemaphoreType.DMA,
            pltpu.SemaphoreType.DMA,
            pltpu.SemaphoreType.DMA,
            pltpu.SemaphoreType.DMA,
            pltpu.SemaphoreType.DMA,
            pltpu.SemaphoreType.DMA,
            pltpu.SemaphoreType.DMA,
            pltpu.SemaphoreType.DMA,
            pltpu.SemaphoreType.DMA,
        ],
    )
    return k(x2, p1)


def kernel(encoded_tokens, pos_table):
    x2 = encoded_tokens.reshape(B, T * D)
    p1 = pos_table.reshape(T * D)
    out = _sc_add(x2, p1)
    return out.reshape(B, T, D)


# speedup vs baseline: 1.0085x; 1.0085x over previous
import jax
import jax.numpy as jnp
from jax import lax
from jax.experimental import pallas as pl
from jax.experimental.pallas import tpu as pltpu
from jax.experimental.pallas import tpu_sc as plsc

B = 4
T = 8192
D = 1024
NC = 2
NS = 16
NW = NC * NS
TPW = T // NW
CH = 4
CHD = CH * D
NCH = TPW // CH
NBUF = 4
UNROLL = 8


def _sc_body(x_hbm, p_hbm, out_hbm, *scratch):
    xbufs = scratch[0:NBUF]
    pbufs = scratch[NBUF:2 * NBUF]
    sxs = scratch[2 * NBUF:3 * NBUF]
    sps = scratch[3 * NBUF:4 * NBUF]
    sos = scratch[4 * NBUF:5 * NBUF]

    wid = lax.axis_index("s") * NC + lax.axis_index("c")
    base = wid * TPW * D

    def start_in(j, r):
        off = base + j * CHD
        pltpu.async_copy(p_hbm.at[pl.ds(off, CHD)], pbufs[r], sps[r])
        pltpu.async_copy(x_hbm.at[:, pl.ds(off, CHD)], xbufs[r], sxs[r])

    def wait_in(r):
        pltpu.make_async_copy(p_hbm.at[pl.ds(0, CHD)], pbufs[r], sps[r]).wait()
        pltpu.make_async_copy(x_hbm.at[:, pl.ds(0, CHD)], xbufs[r], sxs[r]).wait()

    def start_out(j, r):
        off = base + j * CHD
        pltpu.async_copy(xbufs[r], out_hbm.at[:, pl.ds(off, CHD)], sos[r])

    def wait_out(r):
        pltpu.make_async_copy(xbufs[r], out_hbm.at[:, pl.ds(0, CHD)], sos[r]).wait()

    def compute(r):
        xb, pb = xbufs[r], pbufs[r]

        @plsc.parallel_loop(0, CHD, 16, unroll=UNROLL)
        def _(o):
            pv = pb[pl.ds(o, 16)]
            for b in range(B):
                plsc.addupdate(xb.at[b, pl.ds(o, 16)], pv)

    for j in range(NBUF - 1):
        start_in(j, j)

    def group(g, carry):
        for k in range(NBUF):
            j = g * NBUF + k
            wait_in(k)
            compute(k)
            start_out(j, k)
            rn = (k + NBUF - 1) % NBUF

            @pl.when(j > 0)
            def _():
                wait_out(rn)

            @pl.when(j + NBUF - 1 < NCH)
            def _():
                start_in(j + NBUF - 1, rn)

        return carry

    lax.fori_loop(0, NCH // NBUF, group, 0)

    wait_out((NCH - 1) % NBUF)


def _sc_add(x2, p1):
    mesh = plsc.VectorSubcoreMesh(core_axis_name="c", subcore_axis_name="s")
    k = pl.kernel(
        _sc_body,
        out_type=jax.ShapeDtypeStruct((B, T * D), jnp.float32),
        mesh=mesh,
        scratch_types=(
            [pltpu.VMEM((B, CHD), jnp.float32) for _ in range(NBUF)]
            + [pltpu.VMEM((CHD,), jnp.float32) for _ in range(NBUF)]
            + [pltpu.SemaphoreType.DMA for _ in range(3 * NBUF)]
        ),
    )
    return k(x2, p1)


def kernel(encoded_tokens, pos_table):
    x2 = encoded_tokens.reshape(B, T * D)
    p1 = pos_table.reshape(T * D)
    out = _sc_add(x2, p1)
    return out.reshape(B, T, D)


# speedup vs baseline: 2.1667x; 2.1484x over previous
import jax
import jax.numpy as jnp
from jax import lax
from jax.experimental import pallas as pl
from jax.experimental.pallas import tpu as pltpu
from jax.experimental.pallas import tpu_sc as plsc

B = 4
T = 8192
D = 1024
NC = 2
NS = 16
NW = NC * NS
TPW = T // NW
CH = 8
NCH = TPW // CH
NBUF = 2
UNROLL = 4


def _sc_body(x_hbm, p_hbm, out_hbm, *scratch):
    xbufs = scratch[0:NBUF]
    pbufs = scratch[NBUF:2 * NBUF]
    sxs = scratch[2 * NBUF:3 * NBUF]
    sps = scratch[3 * NBUF:4 * NBUF]
    sos = scratch[4 * NBUF:5 * NBUF]

    wid = lax.axis_index("s") * NC + lax.axis_index("c")
    tok0 = wid * TPW

    def start_in(j, r):
        t0 = tok0 + j * CH
        pltpu.async_copy(p_hbm.at[pl.ds(t0, CH), :], pbufs[r], sps[r])
        pltpu.async_copy(x_hbm.at[:, pl.ds(t0, CH), :], xbufs[r], sxs[r])

    def wait_in(r):
        pltpu.make_async_copy(p_hbm.at[pl.ds(0, CH), :], pbufs[r], sps[r]).wait()
        pltpu.make_async_copy(x_hbm.at[:, pl.ds(0, CH), :], xbufs[r], sxs[r]).wait()

    def start_out(j, r):
        t0 = tok0 + j * CH
        pltpu.async_copy(xbufs[r], out_hbm.at[:, pl.ds(t0, CH), :], sos[r])

    def wait_out(r):
        pltpu.make_async_copy(xbufs[r], out_hbm.at[:, pl.ds(0, CH), :], sos[r]).wait()

    def compute(r):
        xb, pb = xbufs[r], pbufs[r]

        @plsc.parallel_loop(0, CH * D, 16, unroll=UNROLL)
        def _(o):
            c = o >> 10
            dd = pl.multiple_of(o & (D - 1), 16)
            pv = pb[c, pl.ds(dd, 16)]
            for b in range(B):
                plsc.addupdate(xb.at[b, c, pl.ds(dd, 16)], pv)

    for j in range(NBUF - 1):
        start_in(j, j)

    def group(g, carry):
        for k in range(NBUF):
            j = g * NBUF + k
            wait_in(k)
            compute(k)
            start_out(j, k)
            rn = (k + NBUF - 1) % NBUF

            @pl.when(j > 0)
            def _():
                wait_out(rn)

            @pl.when(j + NBUF - 1 < NCH)
            def _():
                start_in(j + NBUF - 1, rn)

        return carry

    lax.fori_loop(0, NCH // NBUF, group, 0)

    wait_out((NCH - 1) % NBUF)


def _sc_add(x, p):
    mesh = plsc.VectorSubcoreMesh(core_axis_name="c", subcore_axis_name="s")
    k = pl.kernel(
        _sc_body,
        out_type=jax.ShapeDtypeStruct((B, T, D), jnp.float32),
        mesh=mesh,
        compiler_params=pltpu.CompilerParams(use_tc_tiling_on_sc=True),
        scratch_types=(
            [pltpu.VMEM((B, CH, D), jnp.float32) for _ in range(NBUF)]
            + [pltpu.VMEM((CH, D), jnp.float32) for _ in range(NBUF)]
            + [pltpu.SemaphoreType.DMA for _ in range(3 * NBUF)]
        ),
    )
    return k(x, p)


def kernel(encoded_tokens, pos_table):
    return _sc_add(encoded_tokens, pos_table)


# speedup vs baseline: 2.7415x; 1.2653x over previous
import jax
import jax.numpy as jnp
from jax import lax
from jax.experimental import pallas as pl
from jax.experimental.pallas import tpu as pltpu
from jax.experimental.pallas import tpu_sc as plsc

B = 4
T = 8192
D = 1024
NC = 2
NS = 16
NW = NC * NS
TPW = T // NW
CH = 8
NCH = TPW // CH
NBUF = 3
UNROLL = 4


def _sc_body(x_hbm, p_hbm, out_hbm, *scratch):
    xbufs = scratch[0:NBUF]
    pbufs = scratch[NBUF:2 * NBUF]
    sxs = scratch[2 * NBUF:3 * NBUF]
    sps = scratch[3 * NBUF:4 * NBUF]
    sos = scratch[4 * NBUF:5 * NBUF]

    wid = lax.axis_index("s") * NC + lax.axis_index("c")
    tok0 = wid * TPW

    def start_in(j, r):
        t0 = tok0 + j * CH
        pltpu.async_copy(p_hbm.at[pl.ds(t0, CH), :], pbufs[r], sps[r])
        pltpu.async_copy(x_hbm.at[:, pl.ds(t0, CH), :], xbufs[r], sxs[r])

    def wait_in(r):
        pltpu.make_async_copy(p_hbm.at[pl.ds(0, CH), :], pbufs[r], sps[r]).wait()
        pltpu.make_async_copy(x_hbm.at[:, pl.ds(0, CH), :], xbufs[r], sxs[r]).wait()

    def start_out(j, r):
        t0 = tok0 + j * CH
        pltpu.async_copy(xbufs[r], out_hbm.at[:, pl.ds(t0, CH), :], sos[r])

    def wait_out(r):
        pltpu.make_async_copy(xbufs[r], out_hbm.at[:, pl.ds(0, CH), :], sos[r]).wait()

    def compute(r):
        xb, pb = xbufs[r], pbufs[r]

        @plsc.parallel_loop(0, CH * D, 16, unroll=UNROLL)
        def _(o):
            c = o >> 10
            dd = pl.multiple_of(o & (D - 1), 16)
            pv = pb[c, pl.ds(dd, 16)]
            for b in range(B):
                plsc.addupdate(xb.at[b, c, pl.ds(dd, 16)], pv)

    for j in range(NBUF - 1):
        start_in(j, j)

    def group(g, carry):
        for k in range(NBUF):
            j = g * NBUF + k
            wait_in(k)
            compute(k)
            start_out(j, k)
            rn = (k + NBUF - 1) % NBUF

            @pl.when(j > 0)
            def _():
                wait_out(rn)

            @pl.when(j + NBUF - 1 < NCH)
            def _():
                start_in(j + NBUF - 1, rn)

        return carry

    lax.fori_loop(0, NCH // NBUF, group, 0)

    for j in range((NCH // NBUF) * NBUF, NCH):
        k = j % NBUF
        wait_in(k)
        compute(k)
        start_out(j, k)
        rn = (k + NBUF - 1) % NBUF
        wait_out(rn)
        if j + NBUF - 1 < NCH:
            start_in(j + NBUF - 1, rn)

    wait_out((NCH - 1) % NBUF)


def _sc_add(x, p):
    mesh = plsc.VectorSubcoreMesh(core_axis_name="c", subcore_axis_name="s")
    k = pl.kernel(
        _sc_body,
        out_type=jax.ShapeDtypeStruct((B, T, D), jnp.float32),
        mesh=mesh,
        compiler_params=pltpu.CompilerParams(use_tc_tiling_on_sc=True),
        scratch_types=(
            [pltpu.VMEM((B, CH, D), jnp.float32) for _ in range(NBUF)]
            + [pltpu.VMEM((CH, D), jnp.float32) for _ in range(NBUF)]
            + [pltpu.SemaphoreType.DMA for _ in range(3 * NBUF)]
        ),
    )
    return k(x, p)


def kernel(encoded_tokens, pos_table):
    return _sc_add(encoded_tokens, pos_table)


# speedup vs baseline: 2.8314x; 1.0328x over previous
import jax
import jax.numpy as jnp
from jax import lax
from jax.experimental import pallas as pl
from jax.experimental.pallas import tpu as pltpu
from jax.experimental.pallas import tpu_sc as plsc

B = 4
T = 8192
D = 1024
NC = 2
NS = 16
NW = NC * NS
TPW = T // NW
CH = 8
DC = 512
ND = D // DC
NT = TPW // CH
NCH = NT * ND
NBUF = 6
UNROLL = 4
_LOG_ND = ND.bit_length() - 1
_LOG_DC = DC.bit_length() - 1


def _sc_body(x_hbm, p_hbm, out_hbm, *scratch):
    xbufs = scratch[0:NBUF]
    pbufs = scratch[NBUF:2 * NBUF]
    sxs = scratch[2 * NBUF:3 * NBUF]
    sps = scratch[3 * NBUF:4 * NBUF]
    sos = scratch[4 * NBUF:5 * NBUF]

    wid = lax.axis_index("s") * NC + lax.axis_index("c")
    tok0 = wid * TPW

    def offs(j):
        t0 = pl.multiple_of(tok0 + (j >> _LOG_ND) * CH, CH)
        d0 = (j & (ND - 1)) << _LOG_DC
        if not isinstance(d0, int):
            d0 = pl.multiple_of(d0, DC)
        return t0, d0

    def start_in(j, r):
        t0, d0 = offs(j)
        pltpu.async_copy(p_hbm.at[pl.ds(t0, CH), pl.ds(d0, DC)], pbufs[r], sps[r])
        pltpu.async_copy(x_hbm.at[:, pl.ds(t0, CH), pl.ds(d0, DC)], xbufs[r], sxs[r])

    def wait_in(r):
        pltpu.make_async_copy(
            p_hbm.at[pl.ds(0, CH), pl.ds(0, DC)], pbufs[r], sps[r]).wait()
        pltpu.make_async_copy(
            x_hbm.at[:, pl.ds(0, CH), pl.ds(0, DC)], xbufs[r], sxs[r]).wait()

    def start_out(j, r):
        t0, d0 = offs(j)
        pltpu.async_copy(xbufs[r], out_hbm.at[:, pl.ds(t0, CH), pl.ds(d0, DC)], sos[r])

    def wait_out(r):
        pltpu.make_async_copy(
            xbufs[r], out_hbm.at[:, pl.ds(0, CH), pl.ds(0, DC)], sos[r]).wait()

    def compute(r):
        xb, pb = xbufs[r], pbufs[r]

        @plsc.parallel_loop(0, CH * DC, 16, unroll=UNROLL)
        def _(o):
            c = o >> _LOG_DC
            dd = pl.multiple_of(o & (DC - 1), 16)
            pv = pb[c, pl.ds(dd, 16)]
            for b in range(B):
                plsc.addupdate(xb.at[b, c, pl.ds(dd, 16)], pv)

    for j in range(NBUF - 1):
        start_in(j, j)

    def position(j, k):
        wait_in(k)
        compute(k)
        start_out(j, k)
        rn = (k + NBUF - 1) % NBUF

        @pl.when(j > 0)
        def _():
            wait_out(rn)

        @pl.when(j + NBUF - 1 < NCH)
        def _():
            start_in(j + NBUF - 1, rn)

    def group(g, carry):
        for k in range(NBUF):
            position(g * NBUF + k, k)
        return carry

    lax.fori_loop(0, NCH // NBUF, group, 0)

    for j in range((NCH // NBUF) * NBUF, NCH):
        position(j, j % NBUF)

    wait_out((NCH - 1) % NBUF)


def _sc_add(x, p):
    mesh = plsc.VectorSubcoreMesh(core_axis_name="c", subcore_axis_name="s")
    k = pl.kernel(
        _sc_body,
        out_type=jax.ShapeDtypeStruct((B, T, D), jnp.float32),
        mesh=mesh,
        compiler_params=pltpu.CompilerParams(use_tc_tiling_on_sc=True),
        scratch_types=(
            [pltpu.VMEM((B, CH, DC), jnp.float32) for _ in range(NBUF)]
            + [pltpu.VMEM((CH, DC), jnp.float32) for _ in range(NBUF)]
            + [pltpu.SemaphoreType.DMA for _ in range(3 * NBUF)]
        ),
    )
    return k(x, p)


def kernel(encoded_tokens, pos_table):
    return _sc_add(encoded_tokens, pos_table)
